# baseline (speedup 1.0000x reference)
import functools

import jax
import jax.numpy as jnp
import numpy as np
from jax import lax
from jax.experimental import pallas as pl
from jax.experimental.pallas import tpu as pltpu
from jax.experimental.pallas import tpu_sc as plsc

SAMPLE_REG = np.float32(200.0 / 10.0)
EPS = np.float32(1e-07)
N_TRAJ = 16384
T_LEN = 200
BUF = 4096
TINY = np.float32(np.finfo(np.float32).tiny)


def _np_rotl(x, r):
    return ((x << np.uint32(r)) | (x >> np.uint32(32 - r))).astype(np.uint32)


def _np_threefry2x32(k0, k1, x0, x1):
    ks = [np.uint32(k0), np.uint32(k1),
          np.uint32(np.uint32(k0) ^ np.uint32(k1) ^ np.uint32(0x1BD11BDA))]
    rots = [(13, 15, 26, 6), (17, 29, 16, 24)]
    x0 = np.uint32(x0 + ks[0])
    x1 = np.uint32(x1 + ks[1])
    for g in range(5):
        for r in rots[g % 2]:
            x0 = np.uint32(x0 + x1)
            x1 = _np_rotl(x1, r)
            x1 = np.uint32(x0 ^ x1)
        x0 = np.uint32(x0 + ks[(g + 1) % 3])
        x1 = np.uint32(x1 + ks[(g + 2) % 3] + np.uint32(g + 1))
    return x0, x1


_KEY0, _KEY1 = (w[0] for w in _np_threefry2x32(
    0, 0, np.zeros(1, np.uint32), np.ones(1, np.uint32)))
_KS2 = np.uint32(_KEY0 ^ _KEY1 ^ np.uint32(0x1BD11BDA))


_ROW_BLK = 2048
_N_ROW_BLKS = N_TRAJ // _ROW_BLK


def _logits_body(rew_ref, out_ref, cum_ref):
    i = pl.program_id(0)
    cum = jnp.sum(rew_ref[...], axis=1, keepdims=True)
    cum_ref[pl.ds(i * _ROW_BLK, _ROW_BLK), :] = cum

    @pl.when(i == _N_ROW_BLKS - 1)
    def _():
        allc = cum_ref[...]
        mn = jnp.min(allc)
        shifted = allc - mn + SAMPLE_REG
        s = jnp.sum(shifted)
        out_ref[...] = jnp.log(shifted / s + EPS)


def _compute_logits(rewards2d):
    return pl.pallas_call(
        _logits_body,
        grid=(_N_ROW_BLKS,),
        in_specs=[pl.BlockSpec((_ROW_BLK, T_LEN), lambda i: (i, 0))],
        out_specs=pl.BlockSpec((N_TRAJ, 1), lambda i: (0, 0)),
        out_shape=jax.ShapeDtypeStruct((N_TRAJ, 1), jnp.float32),
        scratch_shapes=[pltpu.VMEM((N_TRAJ, 1), jnp.float32)],
    )(rewards2d)


_BI = 16
_N_STEPS = BUF // _BI
_NCHUNK = 512
_N_INNER = N_TRAJ // _NCHUNK


def _tf_round(x0, x1, r):
    x0 = x0 + x1
    x1 = (x1 << np.uint32(r)) | (x1 >> np.uint32(32 - r))
    x1 = x0 ^ x1
    return x0, x1


def _sampler_body(logits_ref, out_ref):
    step = pl.program_id(0)
    base_row = step * _BI
    row_iota = lax.broadcasted_iota(jnp.uint32, (_BI, _NCHUNK), 0)
    col_iota = lax.broadcasted_iota(jnp.uint32, (_BI, _NCHUNK), 1)
    k_base = (jnp.uint32(base_row) + row_iota) * jnp.uint32(N_TRAJ) + col_iota

    ks = [jnp.uint32(_KEY0), jnp.uint32(_KEY1), jnp.uint32(_KS2)]
    rots = ((13, 15, 26, 6), (17, 29, 16, 24))

    def inner(c, carry):
        m_run, i_run = carry
        n0 = c * _NCHUNK
        x0 = jnp.full((_BI, _NCHUNK), ks[0], dtype=jnp.uint32)
        x1 = (k_base + n0.astype(jnp.uint32)) + ks[1]
        for g in range(5):
            for r in rots[g % 2]:
                x0, x1 = _tf_round(x0, x1, r)
            x0 = x0 + ks[(g + 1) % 3]
            x1 = x1 + ks[(g + 2) % 3] + jnp.uint32(g + 1)
        bits = x0 ^ x1
        fb = (bits >> jnp.uint32(9)) | jnp.uint32(0x3F800000)
        floats = lax.bitcast_convert_type(fb, jnp.float32) - np.float32(1.0)
        u = jnp.maximum(TINY, floats * (np.float32(1.0) - TINY) + TINY)
        g_noise = -jnp.log(-jnp.log(u))
        v = g_noise + logits_ref[0, pl.ds(n0, _NCHUNK)][None, :]
        n_idx = col_iota.astype(jnp.int32) + n0
        upd = v > m_run
        m_run = jnp.where(upd, v, m_run)
        i_run = jnp.where(upd, n_idx, i_run)
        return m_run, i_run

    m0 = jnp.full((_BI, _NCHUNK), -jnp.inf, dtype=jnp.float32)
    i0 = jnp.full((_BI, _NCHUNK), np.int32(2**30), dtype=jnp.int32)
    m_run, i_run = lax.fori_loop(0, _N_INNER, inner, (m0, i0))
    m_final = jnp.max(m_run, axis=1, keepdims=True)
    idx = jnp.min(jnp.where(m_run == m_final, i_run, np.int32(2**30)), axis=1)
    out_ref[0, 0, :] = idx


def _compute_samples(logits_row):
    out = pl.pallas_call(
        _sampler_body,
        grid=(_N_STEPS,),
        in_specs=[pl.BlockSpec((1, N_TRAJ), lambda i: (0, 0))],
        out_specs=pl.BlockSpec((1, 1, _BI), lambda i: (i, 0, 0)),
        out_shape=jax.ShapeDtypeStruct((_N_STEPS, 1, _BI), jnp.int32),
    )(logits_row)
    return out.reshape(BUF)


_D_ST = T_LEN * 8
_D_AC = T_LEN * 2
_D_RW = T_LEN
_D_OP = T_LEN
_CH = 16


def _sc_gather(samples, st2d, ac2d, rw2d, op2d):
    info = plsc.get_sparse_core_info()
    nw = info.num_cores * info.num_subcores
    bpw = BUF // nw
    nch = bpw // _CH
    mesh = plsc.VectorSubcoreMesh(core_axis_name="c", subcore_axis_name="s")

    @functools.partial(
        pl.kernel,
        mesh=mesh,
        compiler_params=pltpu.CompilerParams(use_tc_tiling_on_sc=False),
        out_type=[
            jax.ShapeDtypeStruct((BUF, _D_ST), jnp.float32),
            jax.ShapeDtypeStruct((BUF, _D_AC), jnp.float32),
            jax.ShapeDtypeStruct((BUF, _D_RW), jnp.float32),
            jax.ShapeDtypeStruct((BUF, _D_OP), jnp.float32),
        ],
        scratch_types=[
            pltpu.VMEM((bpw,), jnp.int32),
            pltpu.VMEM((_CH, _D_ST), jnp.float32),
            pltpu.VMEM((_CH, _D_AC), jnp.float32),
            pltpu.VMEM((_CH, _D_RW), jnp.float32),
            pltpu.VMEM((_CH, _D_OP), jnp.float32),
            pltpu.SemaphoreType.DMA,
        ],
    )
    def gather(samp, st, ac, rw, op, ost, oac, orw, oop,
               idx_v, bst, bac, brw, bop, sem):
        wid = lax.axis_index("s") * info.num_cores + lax.axis_index("c")
        base = wid * bpw
        pltpu.sync_copy(samp.at[pl.ds(base, bpw)], idx_v)
        for c in range(nch):
            isl = idx_v.at[pl.ds(c * _CH, _CH)]
            cps = [
                pltpu.async_copy(st.at[isl], bst, sem),
                pltpu.async_copy(ac.at[isl], bac, sem),
                pltpu.async_copy(rw.at[isl], brw, sem),
                pltpu.async_copy(op.at[isl], bop, sem),
            ]
            for cp in cps:
                cp.wait()
            out_sl = pl.ds(base + c * _CH, _CH)
            pltpu.sync_copy(bst, ost.at[out_sl])
            pltpu.sync_copy(bac, oac.at[out_sl])
            pltpu.sync_copy(brw, orw.at[out_sl])
            pltpu.sync_copy(bop, oop.at[out_sl])

    return gather(samples, st2d, ac2d, rw2d, op2d)


def kernel(new_states, new_actions, new_rewards, new_optim, buffer_size):
    rewards2d = new_rewards.reshape(N_TRAJ, T_LEN)
    logits_col = _compute_logits(rewards2d)
    logits_row = logits_col.reshape(1, N_TRAJ)
    samples = _compute_samples(logits_row)
    ost, oac, orw, oop = _sc_gather(
        samples,
        new_states.reshape(N_TRAJ, _D_ST),
        new_actions.reshape(N_TRAJ, _D_AC),
        rewards2d,
        new_optim,
    )
    return (
        ost.reshape(BUF, T_LEN, 8),
        oac.reshape(BUF, T_LEN, 2),
        orw.reshape(BUF, T_LEN, 1),
        oop,
    )
